# Initial kernel scaffold; baseline (speedup 1.0000x reference)
import jax
import jax.numpy as jnp
from jax.experimental import pallas as pl


def kernel(x, params):
    raise NotImplementedError("write your pallas kernel here")


# speedup vs baseline: 4.5581x; 4.5581x over previous
import functools

import jax
import jax.numpy as jnp
import numpy as np
from jax import lax
from jax.experimental import pallas as pl

NPTS = 1024
NBLK = 4
KNN = 16
DMODEL = 64
BIG = 3.0e38


def _dotd(a, b):
    return jnp.dot(a.astype(jnp.bfloat16), b.astype(jnp.bfloat16),
                   preferred_element_type=jnp.float32)


def _dote(a, b):
    return jnp.dot(a, b, preferred_element_type=jnp.float32,
                   precision=lax.Precision.HIGHEST)


def _full_spec(shape):
    nd = len(shape)
    return pl.BlockSpec(shape, lambda *a, _nd=nd: (0,) * _nd)


def _batch_spec(shape):
    nd = len(shape)
    return pl.BlockSpec((1,) + shape, lambda b, _nd=nd: (b,) + (0,) * _nd)


def _topk_idx(d, keff, n):
    cols = lax.broadcasted_iota(jnp.int32, d.shape, 1)
    out = []
    for _ in range(keff):
        m = jnp.min(d, axis=1, keepdims=True)
        am = jnp.min(jnp.where(d == m, cols, n), axis=1, keepdims=True)
        out.append(am)
        d = jnp.where(cols == am, BIG, d)
    return jnp.concatenate(out, axis=1)


def _sq_dists(src, dst_t):
    s1 = jnp.sum(src * src, axis=1, keepdims=True)
    s2 = jnp.sum(dst_t * dst_t, axis=0, keepdims=True)
    return -2.0 * _dotd(src, dst_t) + s1 + s2


def _edge_gather(idx_c, table, c, kpad, n):
    cols3 = lax.broadcasted_iota(jnp.int32, (c, kpad, n), 2)
    oh = (idx_c[:, :, None] == cols3).astype(jnp.float32)
    oh2 = oh.reshape(c * kpad, n)
    return _dote(oh2, table)


def _repeat_rows(block, c, kpad):
    r = lax.broadcasted_iota(jnp.int32, (c * kpad, c), 0)
    q = lax.broadcasted_iota(jnp.int32, (c * kpad, c), 1)
    p = (r // kpad == q).astype(jnp.float32)
    return _dote(p, block)


def _tlayer_body(n, dp, keff, c, pre_mlp, *refs):
    kpad = max(8, keff)
    nchunk = n // c
    if pre_mlp:
        (x_ref, xyz_ref, xyzt_ref, wm1, bm1, wm2, bm2,
         w1, b1, wq, wk, wv, wd1, bd1, wd2, bd2,
         wg1, bg1, wg2, bg2, w2, b2, o_ref) = refs
        xin = x_ref[0]
        f = _dotd(jax.nn.relu(_dotd(xin, wm1[...]) + bm1[...]),
                  wm2[...]) + bm2[...]
    else:
        (f_ref, xyz_ref, xyzt_ref,
         w1, b1, wq, wk, wv, wd1, bd1, wd2, bd2,
         wg1, bg1, wg2, bg2, w2, b2, o_ref) = refs
        f = f_ref[0]
    xyz = xyz_ref[0]
    xyzt = xyzt_ref[0]

    d = _sq_dists(xyz, xyzt)
    knn = _topk_idx(d, keff, n)
    if kpad > keff:
        knn = jnp.concatenate(
            [knn, jnp.zeros((n, kpad - keff), jnp.int32)], axis=1)

    x1 = _dotd(f, w1[...]) + b1[...]
    q = _dotd(x1, wq[...])
    k = _dotd(x1, wk[...])
    v = _dotd(x1, wv[...])
    tbl = jnp.concatenate([xyz, k, v], axis=1)
    iq = jnp.concatenate([xyz, q], axis=1)

    scale = 1.0 / np.sqrt(DMODEL)
    for ci in range(nchunk):
        sl = slice(ci * c, (ci + 1) * c)
        g = _edge_gather(knn[sl], tbl, c, kpad, n)
        ei = _repeat_rows(iq[sl], c, kpad)
        xyz_j, k_j, v_j = g[:, :3], g[:, 3:67], g[:, 67:]
        xyz_i, q_i = ei[:, :3], ei[:, 3:]
        rel = xyz_i - xyz_j
        e1 = jax.nn.relu(_dotd(rel, wd1[...]) + bd1[...])
        pos = _dotd(e1, wd2[...]) + bd2[...]
        g1 = jax.nn.relu(_dotd(q_i - k_j + pos, wg1[...]) + bg1[...])
        logits = (_dotd(g1, wg2[...]) + bg2[...]) * scale
        l3 = logits.reshape(c, kpad, DMODEL)
        if kpad > keff:
            krow = lax.broadcasted_iota(jnp.int32, (c, kpad, DMODEL), 1)
            l3 = jnp.where(krow < keff, l3, -BIG)
        m3 = jnp.max(l3, axis=1, keepdims=True)
        e3 = jnp.exp(l3 - m3)
        s3 = jnp.sum(e3, axis=1, keepdims=True)
        a3 = e3 / s3
        w3 = (v_j + pos).reshape(c, kpad, DMODEL)
        agg = jnp.sum(a3 * w3, axis=1)
        res = _dotd(agg, w2[...]) + b2[...]
        o_ref[0, sl, :] = res + f[sl]


def _run_tlayer(feat, xyz, p, keff, c, pre_mlp, mlp1=None):
    b, n, dp_in = feat.shape
    dp = p["fc2"]["w"].shape[1]
    xyzt = jnp.swapaxes(xyz, 1, 2)
    ws = []
    if pre_mlp:
        ws += [mlp1[0]["w"], mlp1[0]["b"].reshape(1, -1),
               mlp1[1]["w"], mlp1[1]["b"].reshape(1, -1)]
    ws += [p["fc1"]["w"], p["fc1"]["b"].reshape(1, -1),
           p["w_qs"]["w"], p["w_ks"]["w"], p["w_vs"]["w"],
           p["fc_delta"][0]["w"], p["fc_delta"][0]["b"].reshape(1, -1),
           p["fc_delta"][1]["w"], p["fc_delta"][1]["b"].reshape(1, -1),
           p["fc_gamma"][0]["w"], p["fc_gamma"][0]["b"].reshape(1, -1),
           p["fc_gamma"][1]["w"], p["fc_gamma"][1]["b"].reshape(1, -1),
           p["fc2"]["w"], p["fc2"]["b"].reshape(1, -1)]
    body = functools.partial(_tlayer_body, n, dp, keff, c, pre_mlp)
    return pl.pallas_call(
        body,
        grid=(b,),
        in_specs=[_batch_spec((n, dp_in)), _batch_spec((n, 3)),
                  _batch_spec((3, n))] + [_full_spec(w.shape) for w in ws],
        out_specs=_batch_spec((n, dp)),
        out_shape=jax.ShapeDtypeStruct((b, n, dp), jnp.float32),
    )(feat, xyz, xyzt, *ws)


def _fps_body(n, npoint, xyzt_ref, o_ref):
    xyzt = xyzt_ref[0]
    lane = lax.broadcasted_iota(jnp.int32, (1, n), 1)
    ciota = lax.broadcasted_iota(jnp.int32, (1, npoint), 1)

    def step(i, carry):
        dist, far, cent = carry
        cent = jnp.where(ciota == i, far, cent)
        msk = (lane == far).astype(jnp.float32)
        centroid = jnp.sum(xyzt * msk, axis=1, keepdims=True)
        diff = xyzt - centroid
        dd = jnp.sum(diff * diff, axis=0, keepdims=True)
        dist = jnp.minimum(dist, dd)
        mx = jnp.max(dist)
        far2 = jnp.min(jnp.where(dist == mx, lane, n)).astype(jnp.int32)
        return dist, far2, cent

    init = (jnp.full((1, n), 1e10, jnp.float32), jnp.int32(0),
            jnp.zeros((1, npoint), jnp.int32))
    _, _, cent = lax.fori_loop(0, npoint, step, init)
    o_ref[0] = cent


def _run_fps(xyz, npoint):
    b, n, _ = xyz.shape
    xyzt = jnp.swapaxes(xyz, 1, 2)
    body = functools.partial(_fps_body, n, npoint)
    return pl.pallas_call(
        body,
        grid=(b,),
        in_specs=[_batch_spec((3, n))],
        out_specs=_batch_spec((1, npoint)),
        out_shape=jax.ShapeDtypeStruct((b, 1, npoint), jnp.int32),
    )(xyzt)


def _td_body(n, npoint, din, dout, c, xyz_ref, xyzt_ref, f_ref, fps_ref,
             wd_ref, bd_ref, oxyz_ref, of_ref):
    keff = KNN
    nchunk = npoint // c
    xyz = xyz_ref[0]
    xyzt = xyzt_ref[0]
    f = f_ref[0]
    fps = fps_ref[0]
    wd = wd_ref[...]
    bd = bd_ref[...]

    cols3 = lax.broadcasted_iota(jnp.int32, (1, npoint, n), 2)
    ohf = (fps[:, :, None] == cols3).astype(jnp.float32).reshape(npoint, n)
    new_xyz = _dote(ohf, xyz)

    d = _sq_dists(new_xyz, xyzt)
    knn = _topk_idx(d, keff, n)

    tbl = jnp.concatenate([xyz, f], axis=1)
    for ci in range(nchunk):
        sl = slice(ci * c, (ci + 1) * c)
        g = _edge_gather(knn[sl], tbl, c, keff, n)
        nxi = _repeat_rows(new_xyz[sl], c, keff)
        grouped = jnp.concatenate([g[:, :3] - nxi, g[:, 3:]], axis=1)
        val = jax.nn.relu(_dotd(grouped, wd) + bd)
        v3 = val.reshape(c, keff, dout)
        of_ref[0, sl, :] = jnp.max(v3, axis=1)
    oxyz_ref[0] = new_xyz


def _run_td(xyz, feat, fps_idx, p, npoint, c):
    b, n, _ = xyz.shape
    din = feat.shape[2]
    dout = p["w"].shape[1]
    xyzt = jnp.swapaxes(xyz, 1, 2)
    body = functools.partial(_td_body, n, npoint, din, dout, c)
    return pl.pallas_call(
        body,
        grid=(b,),
        in_specs=[_batch_spec((n, 3)), _batch_spec((3, n)),
                  _batch_spec((n, din)), _batch_spec((1, npoint)),
                  _full_spec(p["w"].shape), _full_spec((1, dout))],
        out_specs=(_batch_spec((npoint, 3)), _batch_spec((npoint, dout))),
        out_shape=(jax.ShapeDtypeStruct((b, npoint, 3), jnp.float32),
                   jax.ShapeDtypeStruct((b, npoint, dout), jnp.float32)),
    )(xyz, xyzt, feat, fps_idx, p["w"], p["b"].reshape(1, -1))


def _head_body(f_ref, wa, ba, wb, bb, wc, bc, o_ref):
    f = f_ref[...]
    g = jnp.mean(f, axis=1)
    h = jax.nn.relu(_dotd(g, wa[...]) + ba[...])
    h = jax.nn.relu(_dotd(h, wb[...]) + bb[...])
    o_ref[...] = _dotd(h, wc[...]) + bc[...]


def _run_head(feat, mlp2):
    b, n, d = feat.shape
    nc = mlp2[2]["w"].shape[1]
    ws = [mlp2[0]["w"], mlp2[0]["b"].reshape(1, -1),
          mlp2[1]["w"], mlp2[1]["b"].reshape(1, -1),
          mlp2[2]["w"], mlp2[2]["b"].reshape(1, -1)]
    return pl.pallas_call(
        _head_body,
        in_specs=[_full_spec((b, n, d))] + [_full_spec(w.shape) for w in ws],
        out_specs=_full_spec((b, nc)),
        out_shape=jax.ShapeDtypeStruct((b, nc), jnp.float32),
    )(feat, *ws)


def kernel(x, params):
    xyz = x[..., :3]
    f = _run_tlayer(x, xyz, params["t0"], KNN, 64, True, mlp1=params["mlp1"])
    chunk_t = {256: 256, 64: 64, 16: 16, 4: 4}
    chunk_d = {256: 64, 64: 64, 16: 16, 4: 4}
    for i in range(NBLK):
        npoint = NPTS // 4 ** (i + 1)
        fps_idx = _run_fps(xyz, npoint)
        xyz, f = _run_td(xyz, f, fps_idx, params["down"][i], npoint,
                         chunk_d[npoint])
        keff = min(KNN, npoint)
        f = _run_tlayer(f, xyz, params["tblocks"][i], keff, chunk_t[npoint],
                        False)
    return _run_head(f, params["mlp2"])


# speedup vs baseline: 7.0409x; 1.5447x over previous
import functools

import jax
import jax.numpy as jnp
import numpy as np
from jax import lax
from jax.experimental import pallas as pl
from jax.experimental.pallas import tpu as pltpu

NPTS = 1024
NBLK = 4
KNN = 16
DMODEL = 64
BIG = 3.0e38


def _dotd(a, b):
    return jnp.dot(a.astype(jnp.bfloat16), b.astype(jnp.bfloat16),
                   preferred_element_type=jnp.float32)


def _dote(a, b):
    return jnp.dot(a, b, preferred_element_type=jnp.float32,
                   precision=lax.Precision.HIGHEST)


def _full_spec(shape):
    nd = len(shape)
    return pl.BlockSpec(shape, lambda *a, _nd=nd: (0,) * _nd)


def _batch_spec(shape):
    nd = len(shape)
    return pl.BlockSpec((1,) + shape, lambda b, _nd=nd: (b,) + (0,) * _nd)


def _topk_idx(d, keff, n):
    cols = lax.broadcasted_iota(jnp.int32, d.shape, 1)
    out = []
    for _ in range(keff):
        m = jnp.min(d, axis=1, keepdims=True)
        am = jnp.min(jnp.where(d == m, cols, n), axis=1, keepdims=True)
        out.append(am)
        d = jnp.where(cols == am, BIG, d)
    return jnp.concatenate(out, axis=1)


def _sq_dists(src, dst_t):
    s1 = jnp.sum(src * src, axis=1, keepdims=True)
    s2 = jnp.sum(dst_t * dst_t, axis=0, keepdims=True)
    return -2.0 * _dotd(src, dst_t) + s1 + s2


def _edge_gather(idx_c, table, c, kpad, n):
    cols3 = lax.broadcasted_iota(jnp.int32, (c, kpad, n), 2)
    oh = (idx_c[:, :, None] == cols3).astype(jnp.float32)
    oh2 = oh.reshape(c * kpad, n)
    return _dote(oh2, table)


def _repeat_rows(block, c, kpad):
    r = lax.broadcasted_iota(jnp.int32, (c * kpad, c), 0)
    q = lax.broadcasted_iota(jnp.int32, (c * kpad, c), 1)
    p = (r // kpad == q).astype(jnp.float32)
    return _dote(p, block)


def _tlayer_body(n, dp, keff, c, pre_mlp, *refs):
    kpad = max(8, keff)
    nchunk = n // c
    if pre_mlp:
        (x_ref, xyz_ref, xyzt_ref, wm1, bm1, wm2, bm2,
         w1, b1, wq, wk, wv, wd1, bd1, wd2, bd2,
         wg1, bg1, wg2, bg2, w2, b2, o_ref) = refs
        xin = x_ref[0]
        f = _dotd(jax.nn.relu(_dotd(xin, wm1[...]) + bm1[...]),
                  wm2[...]) + bm2[...]
    else:
        (f_ref, xyz_ref, xyzt_ref,
         w1, b1, wq, wk, wv, wd1, bd1, wd2, bd2,
         wg1, bg1, wg2, bg2, w2, b2, o_ref) = refs
        f = f_ref[0]
    xyz = xyz_ref[0]
    xyzt = xyzt_ref[0]

    d = _sq_dists(xyz, xyzt)
    knn = _topk_idx(d, keff, n)
    if kpad > keff:
        knn = jnp.concatenate(
            [knn, jnp.zeros((n, kpad - keff), jnp.int32)], axis=1)

    x1 = _dotd(f, w1[...]) + b1[...]
    q = _dotd(x1, wq[...])
    k = _dotd(x1, wk[...])
    v = _dotd(x1, wv[...])
    tbl = jnp.concatenate([xyz, k, v], axis=1)
    iq = jnp.concatenate([xyz, q], axis=1)

    scale = 1.0 / np.sqrt(DMODEL)
    for ci in range(nchunk):
        sl = slice(ci * c, (ci + 1) * c)
        g = _edge_gather(knn[sl], tbl, c, kpad, n)
        ei = _repeat_rows(iq[sl], c, kpad)
        xyz_j, k_j, v_j = g[:, :3], g[:, 3:67], g[:, 67:]
        xyz_i, q_i = ei[:, :3], ei[:, 3:]
        rel = xyz_i - xyz_j
        e1 = jax.nn.relu(_dotd(rel, wd1[...]) + bd1[...])
        pos = _dotd(e1, wd2[...]) + bd2[...]
        g1 = jax.nn.relu(_dotd(q_i - k_j + pos, wg1[...]) + bg1[...])
        logits = (_dotd(g1, wg2[...]) + bg2[...]) * scale
        l3 = logits.reshape(c, kpad, DMODEL)
        if kpad > keff:
            krow = lax.broadcasted_iota(jnp.int32, (c, kpad, DMODEL), 1)
            l3 = jnp.where(krow < keff, l3, -BIG)
        m3 = jnp.max(l3, axis=1, keepdims=True)
        e3 = jnp.exp(l3 - m3)
        s3 = jnp.sum(e3, axis=1, keepdims=True)
        a3 = e3 / s3
        w3 = (v_j + pos).reshape(c, kpad, DMODEL)
        agg = jnp.sum(a3 * w3, axis=1)
        res = _dotd(agg, w2[...]) + b2[...]
        o_ref[0, sl, :] = res + f[sl]


def _run_tlayer(feat, xyz, p, keff, c, pre_mlp, mlp1=None):
    b, n, dp_in = feat.shape
    dp = p["fc2"]["w"].shape[1]
    xyzt = jnp.swapaxes(xyz, 1, 2)
    ws = []
    if pre_mlp:
        ws += [mlp1[0]["w"], mlp1[0]["b"].reshape(1, -1),
               mlp1[1]["w"], mlp1[1]["b"].reshape(1, -1)]
    ws += [p["fc1"]["w"], p["fc1"]["b"].reshape(1, -1),
           p["w_qs"]["w"], p["w_ks"]["w"], p["w_vs"]["w"],
           p["fc_delta"][0]["w"], p["fc_delta"][0]["b"].reshape(1, -1),
           p["fc_delta"][1]["w"], p["fc_delta"][1]["b"].reshape(1, -1),
           p["fc_gamma"][0]["w"], p["fc_gamma"][0]["b"].reshape(1, -1),
           p["fc_gamma"][1]["w"], p["fc_gamma"][1]["b"].reshape(1, -1),
           p["fc2"]["w"], p["fc2"]["b"].reshape(1, -1)]
    body = functools.partial(_tlayer_body, n, dp, keff, c, pre_mlp)
    return pl.pallas_call(
        body,
        grid=(b,),
        in_specs=[_batch_spec((n, dp_in)), _batch_spec((n, 3)),
                  _batch_spec((3, n))] + [_full_spec(w.shape) for w in ws],
        out_specs=_batch_spec((n, dp)),
        out_shape=jax.ShapeDtypeStruct((b, n, dp), jnp.float32),
    )(feat, xyz, xyzt, *ws)


def _fps_body(b, n, npoint, xyzt_ref, o_ref, ci_ref):
    xc = xyzt_ref[:, 0, :]
    yc = xyzt_ref[:, 1, :]
    zc = xyzt_ref[:, 2, :]
    lane = lax.broadcasted_iota(jnp.int32, (b, n), 1)
    ci_ref[...] = lax.broadcasted_iota(jnp.int32, (b, npoint), 1)
    ciota = ci_ref[...]

    def step(i, carry):
        dist, far, cent = carry
        cent = cent + far * (ciota == i).astype(jnp.int32)
        msk = (lane == far).astype(jnp.float32)
        ccx = jnp.sum(xc * msk, axis=1, keepdims=True)
        ccy = jnp.sum(yc * msk, axis=1, keepdims=True)
        ccz = jnp.sum(zc * msk, axis=1, keepdims=True)
        dx = xc - ccx
        dy = yc - ccy
        dz = zc - ccz
        dd = (dx * dx + dy * dy) + dz * dz
        dist = jnp.minimum(dist, dd)
        mx = jnp.max(dist, axis=1, keepdims=True)
        far2 = jnp.min(jnp.where(dist == mx, lane, n),
                       axis=1, keepdims=True).astype(jnp.int32)
        return dist, far2, cent

    init = (jnp.full((b, n), 1e10, jnp.float32),
            jnp.zeros((b, 1), jnp.int32),
            jnp.zeros((b, npoint), jnp.int32))
    _, _, cent = lax.fori_loop(0, npoint, step, init)
    o_ref[...] = cent


def _run_fps(xyz, npoint):
    b, n, _ = xyz.shape
    xyzt = jnp.swapaxes(xyz, 1, 2)
    body = functools.partial(_fps_body, b, n, npoint)
    out = pl.pallas_call(
        body,
        in_specs=[_full_spec((b, 3, n))],
        out_specs=_full_spec((b, npoint)),
        out_shape=jax.ShapeDtypeStruct((b, npoint), jnp.int32),
        scratch_shapes=[pltpu.VMEM((b, npoint), jnp.int32)],
    )(xyzt)
    return out.reshape(b, 1, npoint)


def _td_body(n, npoint, din, dout, c, xyz_ref, xyzt_ref, f_ref, fps_ref,
             wd_ref, bd_ref, oxyz_ref, of_ref):
    keff = KNN
    nchunk = npoint // c
    xyz = xyz_ref[0]
    xyzt = xyzt_ref[0]
    f = f_ref[0]
    fps = fps_ref[0]
    wd = wd_ref[...]
    bd = bd_ref[...]

    cols3 = lax.broadcasted_iota(jnp.int32, (1, npoint, n), 2)
    ohf = (fps[:, :, None] == cols3).astype(jnp.float32).reshape(npoint, n)
    new_xyz = _dote(ohf, xyz)

    d = _sq_dists(new_xyz, xyzt)
    knn = _topk_idx(d, keff, n)

    tbl = jnp.concatenate([xyz, f], axis=1)
    for ci in range(nchunk):
        sl = slice(ci * c, (ci + 1) * c)
        g = _edge_gather(knn[sl], tbl, c, keff, n)
        nxi = _repeat_rows(new_xyz[sl], c, keff)
        grouped = jnp.concatenate([g[:, :3] - nxi, g[:, 3:]], axis=1)
        val = jax.nn.relu(_dotd(grouped, wd) + bd)
        v3 = val.reshape(c, keff, dout)
        of_ref[0, sl, :] = jnp.max(v3, axis=1)
    oxyz_ref[0] = new_xyz


def _run_td(xyz, feat, fps_idx, p, npoint, c):
    b, n, _ = xyz.shape
    din = feat.shape[2]
    dout = p["w"].shape[1]
    xyzt = jnp.swapaxes(xyz, 1, 2)
    body = functools.partial(_td_body, n, npoint, din, dout, c)
    return pl.pallas_call(
        body,
        grid=(b,),
        in_specs=[_batch_spec((n, 3)), _batch_spec((3, n)),
                  _batch_spec((n, din)), _batch_spec((1, npoint)),
                  _full_spec(p["w"].shape), _full_spec((1, dout))],
        out_specs=(_batch_spec((npoint, 3)), _batch_spec((npoint, dout))),
        out_shape=(jax.ShapeDtypeStruct((b, npoint, 3), jnp.float32),
                   jax.ShapeDtypeStruct((b, npoint, dout), jnp.float32)),
    )(xyz, xyzt, feat, fps_idx, p["w"], p["b"].reshape(1, -1))


def _head_body(f_ref, wa, ba, wb, bb, wc, bc, o_ref):
    f = f_ref[...]
    g = jnp.mean(f, axis=1)
    h = jax.nn.relu(_dotd(g, wa[...]) + ba[...])
    h = jax.nn.relu(_dotd(h, wb[...]) + bb[...])
    o_ref[...] = _dotd(h, wc[...]) + bc[...]


def _run_head(feat, mlp2):
    b, n, d = feat.shape
    nc = mlp2[2]["w"].shape[1]
    ws = [mlp2[0]["w"], mlp2[0]["b"].reshape(1, -1),
          mlp2[1]["w"], mlp2[1]["b"].reshape(1, -1),
          mlp2[2]["w"], mlp2[2]["b"].reshape(1, -1)]
    return pl.pallas_call(
        _head_body,
        in_specs=[_full_spec((b, n, d))] + [_full_spec(w.shape) for w in ws],
        out_specs=_full_spec((b, nc)),
        out_shape=jax.ShapeDtypeStruct((b, nc), jnp.float32),
    )(feat, *ws)


def kernel(x, params):
    xyz = x[..., :3]
    f = _run_tlayer(x, xyz, params["t0"], KNN, 64, True, mlp1=params["mlp1"])
    chunk_t = {256: 256, 64: 64, 16: 16, 4: 4}
    chunk_d = {256: 64, 64: 64, 16: 16, 4: 4}
    for i in range(NBLK):
        npoint = NPTS // 4 ** (i + 1)
        fps_idx = _run_fps(xyz, npoint)
        xyz, f = _run_td(xyz, f, fps_idx, params["down"][i], npoint,
                         chunk_d[npoint])
        keff = min(KNN, npoint)
        f = _run_tlayer(f, xyz, params["tblocks"][i], keff, chunk_t[npoint],
                        False)
    return _run_head(f, params["mlp2"])


# speedup vs baseline: 10.5708x; 1.5013x over previous
import functools

import jax
import jax.numpy as jnp
import numpy as np
from jax import lax
from jax.experimental import pallas as pl
from jax.experimental.pallas import tpu as pltpu

NPTS = 1024
NBLK = 4
KNN = 16
DMODEL = 64
BIG = 3.0e38


def _dotd(a, b):
    return jnp.dot(a.astype(jnp.bfloat16), b.astype(jnp.bfloat16),
                   preferred_element_type=jnp.float32)


def _dote(a, b):
    return jnp.dot(a, b, preferred_element_type=jnp.float32,
                   precision=lax.Precision.HIGHEST)


def _full_spec(shape):
    nd = len(shape)
    return pl.BlockSpec(shape, lambda *a, _nd=nd: (0,) * _nd)


def _batch_spec(shape):
    nd = len(shape)
    return pl.BlockSpec((1,) + shape, lambda b, _nd=nd: (b,) + (0,) * _nd)


def _topk_idx(d, keff, n):
    cols = lax.broadcasted_iota(jnp.int32, d.shape, 1)
    out = []
    for _ in range(keff):
        m = jnp.min(d, axis=1, keepdims=True)
        am = jnp.min(jnp.where(d == m, cols, n), axis=1, keepdims=True)
        out.append(am)
        d = jnp.where(cols == am, BIG, d)
    return jnp.concatenate(out, axis=1)


def _sq_dists(src, dst_t):
    s1 = jnp.sum(src * src, axis=1, keepdims=True)
    s2 = jnp.sum(dst_t * dst_t, axis=0, keepdims=True)
    return -2.0 * _dotd(src, dst_t) + s1 + s2


def _split3(table):
    hi = table.astype(jnp.bfloat16)
    r1 = table - hi.astype(jnp.float32)
    mid = r1.astype(jnp.bfloat16)
    lo = (r1 - mid.astype(jnp.float32)).astype(jnp.bfloat16)
    return jnp.concatenate([hi, mid, lo], axis=1)


def _edge_gather(idx_c, table3, c, kpad, n, w):
    cols3 = lax.broadcasted_iota(jnp.int32, (c, kpad, n), 2)
    oh = (idx_c[:, :, None] == cols3).astype(jnp.bfloat16)
    oh2 = oh.reshape(c * kpad, n)
    g3 = jnp.dot(oh2, table3, preferred_element_type=jnp.float32)
    return (g3[:, :w] + g3[:, w:2 * w]) + g3[:, 2 * w:]


def _bcast_rows(block, c, kpad):
    cc, w = block.shape
    return jnp.broadcast_to(block[:, None, :], (cc, kpad, w)).reshape(
        cc * kpad, w)


def _tlayer_body(n, dp, keff, c, pre_mlp, *refs):
    kpad = max(8, keff)
    nchunk = n // c
    if pre_mlp:
        (x_ref, xyz_ref, xyzt_ref, wm1, bm1, wm2, bm2,
         w1, b1, wq, wk, wv, wd1, bd1, wd2, bd2,
         wg1, bg1, wg2, bg2, w2, b2, o_ref) = refs
        xin = x_ref[0]
        f = _dotd(jax.nn.relu(_dotd(xin, wm1[...]) + bm1[...]),
                  wm2[...]) + bm2[...]
    else:
        (f_ref, xyz_ref, xyzt_ref,
         w1, b1, wq, wk, wv, wd1, bd1, wd2, bd2,
         wg1, bg1, wg2, bg2, w2, b2, o_ref) = refs
        f = f_ref[0]
    xyz = xyz_ref[0]
    xyzt = xyzt_ref[0]

    d = _sq_dists(xyz, xyzt)
    knn = _topk_idx(d, keff, n)
    if kpad > keff:
        knn = jnp.concatenate(
            [knn, jnp.zeros((n, kpad - keff), jnp.int32)], axis=1)

    x1 = _dotd(f, w1[...]) + b1[...]
    q = _dotd(x1, wq[...])
    k = _dotd(x1, wk[...])
    v = _dotd(x1, wv[...])
    tbl3 = _split3(jnp.concatenate([xyz, k, v], axis=1))
    iq = jnp.concatenate([xyz, q], axis=1)

    scale = 1.0 / np.sqrt(DMODEL)
    for ci in range(nchunk):
        sl = slice(ci * c, (ci + 1) * c)
        g = _edge_gather(knn[sl], tbl3, c, kpad, n, 131)
        ei = _bcast_rows(iq[sl], c, kpad)
        xyz_j, k_j, v_j = g[:, :3], g[:, 3:67], g[:, 67:]
        xyz_i, q_i = ei[:, :3], ei[:, 3:]
        rel = xyz_i - xyz_j
        e1 = jax.nn.relu(_dotd(rel, wd1[...]) + bd1[...])
        pos = _dotd(e1, wd2[...]) + bd2[...]
        g1 = jax.nn.relu(_dotd(q_i - k_j + pos, wg1[...]) + bg1[...])
        logits = (_dotd(g1, wg2[...]) + bg2[...]) * scale
        l3 = logits.reshape(c, kpad, DMODEL)
        if kpad > keff:
            krow = lax.broadcasted_iota(jnp.int32, (c, kpad, DMODEL), 1)
            l3 = jnp.where(krow < keff, l3, -BIG)
        m3 = jnp.max(l3, axis=1, keepdims=True)
        e3 = jnp.exp(l3 - m3)
        s3 = jnp.sum(e3, axis=1, keepdims=True)
        a3 = e3 / s3
        w3 = (v_j + pos).reshape(c, kpad, DMODEL)
        agg = jnp.sum(a3 * w3, axis=1)
        res = _dotd(agg, w2[...]) + b2[...]
        o_ref[0, sl, :] = res + f[sl]


def _run_tlayer(feat, xyz, p, keff, c, pre_mlp, mlp1=None):
    b, n, dp_in = feat.shape
    dp = p["fc2"]["w"].shape[1]
    xyzt = jnp.swapaxes(xyz, 1, 2)
    ws = []
    if pre_mlp:
        ws += [mlp1[0]["w"], mlp1[0]["b"].reshape(1, -1),
               mlp1[1]["w"], mlp1[1]["b"].reshape(1, -1)]
    ws += [p["fc1"]["w"], p["fc1"]["b"].reshape(1, -1),
           p["w_qs"]["w"], p["w_ks"]["w"], p["w_vs"]["w"],
           p["fc_delta"][0]["w"], p["fc_delta"][0]["b"].reshape(1, -1),
           p["fc_delta"][1]["w"], p["fc_delta"][1]["b"].reshape(1, -1),
           p["fc_gamma"][0]["w"], p["fc_gamma"][0]["b"].reshape(1, -1),
           p["fc_gamma"][1]["w"], p["fc_gamma"][1]["b"].reshape(1, -1),
           p["fc2"]["w"], p["fc2"]["b"].reshape(1, -1)]
    body = functools.partial(_tlayer_body, n, dp, keff, c, pre_mlp)
    return pl.pallas_call(
        body,
        grid=(b,),
        in_specs=[_batch_spec((n, dp_in)), _batch_spec((n, 3)),
                  _batch_spec((3, n))] + [_full_spec(w.shape) for w in ws],
        out_specs=_batch_spec((n, dp)),
        out_shape=jax.ShapeDtypeStruct((b, n, dp), jnp.float32),
    )(feat, xyz, xyzt, *ws)


def _fps_body(b, n, npoint, xyzt_ref, o_ref, ci_ref):
    xc = xyzt_ref[:, 0, :]
    yc = xyzt_ref[:, 1, :]
    zc = xyzt_ref[:, 2, :]
    lane = lax.broadcasted_iota(jnp.int32, (b, n), 1)
    ci_ref[...] = lax.broadcasted_iota(jnp.int32, (b, npoint), 1)
    ciota = ci_ref[...]

    def step(i, carry):
        dist, far, cent = carry
        cent = cent + far * (ciota == i).astype(jnp.int32)
        msk = (lane == far).astype(jnp.float32)
        ccx = jnp.sum(xc * msk, axis=1, keepdims=True)
        ccy = jnp.sum(yc * msk, axis=1, keepdims=True)
        ccz = jnp.sum(zc * msk, axis=1, keepdims=True)
        dx = xc - ccx
        dy = yc - ccy
        dz = zc - ccz
        dd = (dx * dx + dy * dy) + dz * dz
        dist = jnp.minimum(dist, dd)
        mx = jnp.max(dist, axis=1, keepdims=True)
        far2 = jnp.min(jnp.where(dist == mx, lane, n),
                       axis=1, keepdims=True).astype(jnp.int32)
        return dist, far2, cent

    init = (jnp.full((b, n), 1e10, jnp.float32),
            jnp.zeros((b, 1), jnp.int32),
            jnp.zeros((b, npoint), jnp.int32))
    _, _, cent = lax.fori_loop(0, npoint, step, init)
    o_ref[...] = cent


def _run_fps(xyz, npoint):
    b, n, _ = xyz.shape
    xyzt = jnp.swapaxes(xyz, 1, 2)
    body = functools.partial(_fps_body, b, n, npoint)
    out = pl.pallas_call(
        body,
        in_specs=[_full_spec((b, 3, n))],
        out_specs=_full_spec((b, npoint)),
        out_shape=jax.ShapeDtypeStruct((b, npoint), jnp.int32),
        scratch_shapes=[pltpu.VMEM((b, npoint), jnp.int32)],
    )(xyzt)
    return out.reshape(b, 1, npoint)


def _td_body(n, npoint, din, dout, c, xyz_ref, xyzt_ref, f_ref, fps_ref,
             wd_ref, bd_ref, oxyz_ref, of_ref):
    keff = KNN
    nchunk = npoint // c
    xyz = xyz_ref[0]
    xyzt = xyzt_ref[0]
    f = f_ref[0]
    fps = fps_ref[0]
    wd = wd_ref[...]
    bd = bd_ref[...]

    cols3 = lax.broadcasted_iota(jnp.int32, (1, npoint, n), 2)
    ohf = (fps[:, :, None] == cols3).astype(jnp.float32).reshape(npoint, n)
    new_xyz = _dote(ohf, xyz)

    d = _sq_dists(new_xyz, xyzt)
    knn = _topk_idx(d, keff, n)

    tbl3 = _split3(jnp.concatenate([xyz, f], axis=1))
    for ci in range(nchunk):
        sl = slice(ci * c, (ci + 1) * c)
        g = _edge_gather(knn[sl], tbl3, c, keff, n, 3 + din)
        nxi = _bcast_rows(new_xyz[sl], c, keff)
        grouped = jnp.concatenate([g[:, :3] - nxi, g[:, 3:]], axis=1)
        val = jax.nn.relu(_dotd(grouped, wd) + bd)
        v3 = val.reshape(c, keff, dout)
        of_ref[0, sl, :] = jnp.max(v3, axis=1)
    oxyz_ref[0] = new_xyz


def _run_td(xyz, feat, fps_idx, p, npoint, c):
    b, n, _ = xyz.shape
    din = feat.shape[2]
    dout = p["w"].shape[1]
    xyzt = jnp.swapaxes(xyz, 1, 2)
    body = functools.partial(_td_body, n, npoint, din, dout, c)
    return pl.pallas_call(
        body,
        grid=(b,),
        in_specs=[_batch_spec((n, 3)), _batch_spec((3, n)),
                  _batch_spec((n, din)), _batch_spec((1, npoint)),
                  _full_spec(p["w"].shape), _full_spec((1, dout))],
        out_specs=(_batch_spec((npoint, 3)), _batch_spec((npoint, dout))),
        out_shape=(jax.ShapeDtypeStruct((b, npoint, 3), jnp.float32),
                   jax.ShapeDtypeStruct((b, npoint, dout), jnp.float32)),
    )(xyz, xyzt, feat, fps_idx, p["w"], p["b"].reshape(1, -1))


def _head_body(f_ref, wa, ba, wb, bb, wc, bc, o_ref):
    f = f_ref[...]
    g = jnp.mean(f, axis=1)
    h = jax.nn.relu(_dotd(g, wa[...]) + ba[...])
    h = jax.nn.relu(_dotd(h, wb[...]) + bb[...])
    o_ref[...] = _dotd(h, wc[...]) + bc[...]


def _run_head(feat, mlp2):
    b, n, d = feat.shape
    nc = mlp2[2]["w"].shape[1]
    ws = [mlp2[0]["w"], mlp2[0]["b"].reshape(1, -1),
          mlp2[1]["w"], mlp2[1]["b"].reshape(1, -1),
          mlp2[2]["w"], mlp2[2]["b"].reshape(1, -1)]
    return pl.pallas_call(
        _head_body,
        in_specs=[_full_spec((b, n, d))] + [_full_spec(w.shape) for w in ws],
        out_specs=_full_spec((b, nc)),
        out_shape=jax.ShapeDtypeStruct((b, nc), jnp.float32),
    )(feat, *ws)


def kernel(x, params):
    xyz = x[..., :3]
    f = _run_tlayer(x, xyz, params["t0"], KNN, 64, True, mlp1=params["mlp1"])
    chunk_t = {256: 256, 64: 64, 16: 16, 4: 4}
    chunk_d = {256: 64, 64: 64, 16: 16, 4: 4}
    for i in range(NBLK):
        npoint = NPTS // 4 ** (i + 1)
        fps_idx = _run_fps(xyz, npoint)
        xyz, f = _run_td(xyz, f, fps_idx, params["down"][i], npoint,
                         chunk_d[npoint])
        keff = min(KNN, npoint)
        f = _run_tlayer(f, xyz, params["tblocks"][i], keff, chunk_t[npoint],
                        False)
    return _run_head(f, params["mlp2"])
